# Initial kernel scaffold; baseline (speedup 1.0000x reference)
import jax
import jax.numpy as jnp
from jax.experimental import pallas as pl


def kernel(x, edge_index, batch, edge_attr, graph_fea, embed, W1, b1, W2, b2):
    raise NotImplementedError("write your pallas kernel here")


# speedup vs baseline: 8.0312x; 8.0312x over previous
import functools

import jax
import jax.numpy as jnp
from jax import lax
from jax.experimental import pallas as pl
from jax.experimental.pallas import tpu as pltpu
from jax.experimental.pallas import tpu_sc as plsc

N = 10000
NP = 10240
E = 320000
D = 128
G = 64
V = 100000
NC, NS, L = 2, 16, 16
NW = NC * NS
RPW = NP // NW
EPW = E // NW
CH = 80
NECH = EPW // CH
RPT = NP // NS

f32 = jnp.float32
i32 = jnp.int32


def _mesh():
    return plsc.VectorSubcoreMesh(
        core_axis_name="c", subcore_axis_name="s",
        num_cores=NC, num_subcores=NS)


def _sc_prep_body(x_ref, col_ref, ew_ref, embed_ref, zcol_ref,
                  h0_ref, degp_ref,
                  idxv, rowsv, colv, ewv, buf16, deg_sh, sem):
    c = lax.axis_index("c")
    s = lax.axis_index("s")
    w = s * NC + c

    pltpu.sync_copy(zcol_ref.at[pl.ds(s * RPT, RPT)],
                    deg_sh.at[pl.ds(s * RPT, RPT)])
    plsc.subcore_barrier()

    def gbody(t, _):
        base = w * RPW + t * CH
        pltpu.sync_copy(x_ref.at[pl.ds(base, CH)], idxv)
        pltpu.async_copy(embed_ref.at[idxv], rowsv, sem).wait()
        pltpu.sync_copy(rowsv, h0_ref.at[pl.ds(base, CH)])
        return 0

    lax.fori_loop(0, RPW // CH, gbody, 0)

    def dbody(j, _):
        base = w * EPW + j * CH
        pltpu.sync_copy(col_ref.at[pl.ds(base, CH)], colv)
        pltpu.sync_copy(ew_ref.at[pl.ds(base, CH)], ewv)

        def fill(i, _):
            sv = plsc.load_gather(ewv, [jnp.full((L,), i, i32)])
            buf16[i, pl.ds(0, L)] = sv
            return 0

        lax.fori_loop(0, CH, fill, 0)
        pltpu.sync_copy(buf16, deg_sh.at[colv], add=True)
        return 0

    lax.fori_loop(0, NECH, dbody, 0)
    plsc.subcore_barrier()
    pltpu.sync_copy(deg_sh.at[pl.ds(s * RPT, RPT)],
                    degp_ref.at[c, pl.ds(s * RPT, RPT)])


@functools.cache
def _sc_prep():
    return pl.kernel(
        _sc_prep_body,
        out_type=(
            jax.ShapeDtypeStruct((NP, D), f32),
            jax.ShapeDtypeStruct((NC, NP, L), f32),
        ),
        mesh=_mesh(),
        compiler_params=pltpu.CompilerParams(needs_layout_passes=False),
        scratch_types=(
            pltpu.VMEM((CH,), i32),
            pltpu.VMEM((CH, D), f32),
            pltpu.VMEM((CH,), i32),
            pltpu.VMEM((CH,), f32),
            pltpu.VMEM((CH, L), f32),
            pltpu.VMEM_SHARED((NP, L), f32),
            pltpu.SemaphoreType.DMA,
        ),
    )


def _sc_edge_pass_body(hs_ref, row_ref, col_ref, ew_ref,
                       accp_ref,
                       rowv, colv, ewv, rowsv, acc_sh, sem):
    c = lax.axis_index("c")
    s = lax.axis_index("s")
    w = s * NC + c

    z16 = jnp.zeros((L,), f32)

    def zfill(i, _):
        for q in range(D // L):
            rowsv[i, pl.ds(q * L, L)] = z16
        return 0

    lax.fori_loop(0, CH, zfill, 0)
    for m in range(RPT // CH):
        pltpu.sync_copy(rowsv, acc_sh.at[pl.ds(s * RPT + m * CH, CH)])
    plsc.subcore_barrier()

    def ebody(j, _):
        base = w * EPW + j * CH
        pltpu.sync_copy(row_ref.at[pl.ds(base, CH)], rowv)
        pltpu.sync_copy(col_ref.at[pl.ds(base, CH)], colv)
        pltpu.sync_copy(ew_ref.at[pl.ds(base, CH)], ewv)
        pltpu.async_copy(hs_ref.at[rowv], rowsv, sem).wait()

        def scale(i, _):
            sv = plsc.load_gather(ewv, [jnp.full((L,), i, i32)])
            for q in range(D // L):
                rowsv[i, pl.ds(q * L, L)] = rowsv[i, pl.ds(q * L, L)] * sv
            return 0

        lax.fori_loop(0, CH, scale, 0)
        pltpu.sync_copy(rowsv, acc_sh.at[colv], add=True)
        return 0

    lax.fori_loop(0, NECH, ebody, 0)
    plsc.subcore_barrier()
    pltpu.sync_copy(acc_sh.at[pl.ds(s * RPT, RPT)],
                    accp_ref.at[c, pl.ds(s * RPT, RPT)])


@functools.cache
def _sc_edge_pass():
    return pl.kernel(
        _sc_edge_pass_body,
        out_type=jax.ShapeDtypeStruct((NC, NP, D), f32),
        mesh=_mesh(),
        compiler_params=pltpu.CompilerParams(needs_layout_passes=False),
        scratch_types=(
            pltpu.VMEM((CH,), i32),
            pltpu.VMEM((CH,), i32),
            pltpu.VMEM((CH,), f32),
            pltpu.VMEM((CH, D), f32),
            pltpu.VMEM_SHARED((NP, D), f32),
            pltpu.SemaphoreType.DMA,
        ),
    )


def _tc_scale_body(degp_ref, h0_ref, dinv_ref, hs0_ref):
    deg = degp_ref[0, :, 0:1] + degp_ref[1, :, 0:1] + 1.0
    dinv = lax.rsqrt(deg)
    dinv_ref[...] = dinv
    hs0_ref[...] = h0_ref[...] * dinv


_tc_scale = pl.pallas_call(
    _tc_scale_body,
    grid=(NP // 128,),
    in_specs=[
        pl.BlockSpec((NC, 128, L), lambda i: (0, i, 0)),
        pl.BlockSpec((128, D), lambda i: (i, 0)),
    ],
    out_specs=[
        pl.BlockSpec((128, 1), lambda i: (i, 0)),
        pl.BlockSpec((128, D), lambda i: (i, 0)),
    ],
    out_shape=[
        jax.ShapeDtypeStruct((NP, 1), f32),
        jax.ShapeDtypeStruct((NP, D), f32),
    ],
)


def _tc_mlp_body(accp_ref, hs0_ref, dinv_ref, w1_ref, b1_ref, w2_ref,
                 hs1_ref):
    dinv = dinv_ref[...]
    p0 = (accp_ref[0] + accp_ref[1] + hs0_ref[...]) * dinv
    h1 = jnp.maximum(
        jnp.dot(p0, w1_ref[...], preferred_element_type=f32) + b1_ref[...],
        0.0)
    t = jnp.dot(h1, w2_ref[...], preferred_element_type=f32)
    hs1_ref[...] = t * dinv


_tc_mlp = pl.pallas_call(
    _tc_mlp_body,
    grid=(NP // 128,),
    in_specs=[
        pl.BlockSpec((NC, 128, D), lambda i: (0, i, 0)),
        pl.BlockSpec((128, D), lambda i: (i, 0)),
        pl.BlockSpec((128, 1), lambda i: (i, 0)),
        pl.BlockSpec((D, 2 * D), lambda i: (0, 0)),
        pl.BlockSpec((1, 2 * D), lambda i: (0, 0)),
        pl.BlockSpec((2 * D, D), lambda i: (0, 0)),
    ],
    out_specs=pl.BlockSpec((128, D), lambda i: (i, 0)),
    out_shape=jax.ShapeDtypeStruct((NP, D), f32),
)


def _tc_final_body(accp_ref, hs1_ref, dinv_ref, b2_ref, batch_ref,
                   out_ref, sums, cnt):
    step = pl.program_id(0)
    h2 = jnp.maximum(
        (accp_ref[0] + accp_ref[1] + hs1_ref[...]) * dinv_ref[...]
        + b2_ref[...],
        0.0)
    bt = batch_ref[...].reshape(1, 128)
    oh = (lax.broadcasted_iota(i32, (G, 128), 0) == bt).astype(f32)
    part = jnp.dot(oh, h2, preferred_element_type=f32)
    pcnt = jnp.sum(oh, axis=1, keepdims=True)

    @pl.when(step == 0)
    def _():
        sums[...] = part
        cnt[...] = pcnt

    @pl.when(step > 0)
    def _():
        sums[...] = sums[...] + part
        cnt[...] = cnt[...] + pcnt

    out_ref[...] = sums[...] / jnp.maximum(cnt[...], 1.0)


_tc_final = pl.pallas_call(
    _tc_final_body,
    grid=(NP // 128,),
    in_specs=[
        pl.BlockSpec((NC, 128, D), lambda i: (0, i, 0)),
        pl.BlockSpec((128, D), lambda i: (i, 0)),
        pl.BlockSpec((128, 1), lambda i: (i, 0)),
        pl.BlockSpec((1, D), lambda i: (0, 0)),
        pl.BlockSpec((128,), lambda i: (i,)),
    ],
    out_specs=pl.BlockSpec((G, D), lambda i: (0, 0)),
    out_shape=jax.ShapeDtypeStruct((G, D), f32),
    scratch_shapes=[pltpu.VMEM((G, D), f32), pltpu.VMEM((G, 1), f32)],
)


def kernel(x, edge_index, batch, edge_attr, graph_fea, embed, W1, b1, W2, b2):
    del graph_fea
    x_pad = jnp.concatenate(
        [x.astype(i32), jnp.zeros((NP - N,), i32)])
    batch_pad = jnp.concatenate(
        [batch.astype(i32), jnp.full((NP - N,), G, i32)])
    row = edge_index[0].astype(i32)
    col = edge_index[1].astype(i32)
    ew1 = edge_attr.astype(f32)
    zcol = jnp.zeros((NP, L), f32)

    h0, degp = _sc_prep()(x_pad, col, ew1, embed.astype(f32), zcol)
    dinv, hs0 = _tc_scale(degp, h0)
    acc1 = _sc_edge_pass()(hs0, row, col, ew1)
    hs1 = _tc_mlp(acc1, hs0, dinv, W1.astype(f32),
                  b1.astype(f32).reshape(1, 2 * D), W2.astype(f32))
    acc2 = _sc_edge_pass()(hs1, row, col, ew1)
    out = _tc_final(acc2, hs1, dinv, b2.astype(f32).reshape(1, D), batch_pad)
    return out
